# baseline (speedup 1.0000x reference)
import functools

import jax
import jax.numpy as jnp
from jax import lax
from jax.experimental import pallas as pl
from jax.experimental.pallas import tpu as pltpu
from jax.experimental.pallas import tpu_sc as plsc

N = 10000
D_IN = 128
D_HID = 256
D_OUT = 128

NC, NS = 2, 16
NW = NC * NS
NPAD = 10240
ROWS_PER_TILE = NPAD // NS
CHUNK = 128
NCHUNKS = 2500
WIN = 80
PHASE = 40
NIDX = PHASE + 8
NPADROWS = 2568
BLK = 400


def _sc_mesh():
    return plsc.VectorSubcoreMesh(core_axis_name="c", subcore_axis_name="s")


def _zero_rows(buf, nrows):
    z = jnp.zeros((16,), jnp.float32)

    @pl.loop(0, nrows)
    def _(i):
        @pl.loop(0, 8)
        def _(j):
            buf[i, pl.ds(j * 16, 16)] = z


def _sc_degree(dst2d):
    cp = pltpu.CompilerParams(needs_layout_passes=False)

    @functools.partial(
        pl.kernel,
        out_type=jax.ShapeDtypeStruct((NC, 1, NPAD), jnp.float32),
        mesh=_sc_mesh(),
        scratch_types=[
            pltpu.VMEM((WIN, CHUNK), jnp.int32),
            pltpu.VMEM((NPAD,), jnp.float32),
            pltpu.VMEM((ROWS_PER_TILE,), jnp.float32),
            pltpu.VMEM_SHARED((NS, NPAD), jnp.float32),
        ],
        compiler_params=cp,
    )
    def k(dst_hbm, out_hbm, idxd, loc, red, shr):
        cid = lax.axis_index("c")
        sid = lax.axis_index("s")
        wid = sid * NC + cid

        pltpu.sync_copy(dst_hbm.at[pl.ds(wid * WIN, WIN)], idxd)

        z = jnp.zeros((16,), jnp.float32)

        @pl.loop(0, NPAD // 16)
        def _(i):
            loc[pl.ds(i * 16, 16)] = z

        one = jnp.ones((16,), jnp.float32)

        @pl.loop(0, WIN)
        def _(i):
            @pl.loop(0, CHUNK // 16)
            def _(j):
                plsc.addupdate_scatter(loc, [idxd[i, pl.ds(j * 16, 16)]], one)

        pltpu.sync_copy(loc, shr.at[sid])
        plsc.subcore_barrier()

        c0 = sid * ROWS_PER_TILE
        pltpu.sync_copy(shr.at[0, pl.ds(c0, ROWS_PER_TILE)], red)

        @pl.loop(1, NS)
        def _(t):
            pltpu.sync_copy(shr.at[t, pl.ds(c0, ROWS_PER_TILE)],
                            loc.at[pl.ds(0, ROWS_PER_TILE)])

            @pl.loop(0, ROWS_PER_TILE // 16)
            def _(v):
                s16 = pl.ds(v * 16, 16)
                red[s16] = red[s16] + loc[s16]

        pltpu.sync_copy(red, out_hbm.at[cid, 0, pl.ds(c0, ROWS_PER_TILE)])

    return k(dst2d)


def _sc_aggregate(g, src2d, dst2d):
    @functools.partial(
        pl.kernel,
        out_type=jax.ShapeDtypeStruct((NC, NPAD, 128), jnp.float32),
        mesh=_sc_mesh(),
        scratch_types=[
            pltpu.VMEM((NIDX, CHUNK), jnp.int32),
            pltpu.VMEM((NIDX, CHUNK), jnp.int32),
            pltpu.VMEM((CHUNK, 128), jnp.float32),
            pltpu.VMEM((CHUNK, 128), jnp.float32),
            pltpu.VMEM_SHARED((NPAD, 128), jnp.float32),
            pltpu.SemaphoreType.DMA,
            pltpu.SemaphoreType.DMA,
            pltpu.SemaphoreType.DMA,
            pltpu.SemaphoreType.DMA,
        ],
    )
    def k(g_hbm, src_hbm, dst_hbm, out_hbm, idxs, idxd, rows0, rows1,
          acc, semg0, semg1, sems0, sems1):
        cid = lax.axis_index("c")
        sid = lax.axis_index("s")
        wid = sid * NC + cid
        q0 = wid * WIN

        _zero_rows(rows0, CHUNK)

        @pl.loop(0, ROWS_PER_TILE // CHUNK)
        def _(j):
            pltpu.async_copy(
                rows0, acc.at[pl.ds(sid * ROWS_PER_TILE + j * CHUNK, CHUNK)],
                sems0,
            )

        @pl.loop(0, ROWS_PER_TILE // CHUNK)
        def _(j):
            pltpu.make_async_copy(
                rows0, acc.at[pl.ds(sid * ROWS_PER_TILE + j * CHUNK, CHUNK)],
                sems0,
            ).wait()

        plsc.subcore_barrier()

        npairs = PHASE // 2

        @pl.loop(0, WIN // PHASE)
        def _(p):
            qp = q0 + PHASE * p
            pltpu.async_copy(src_hbm.at[pl.ds(qp, NIDX)], idxs, semg0)
            pltpu.async_copy(dst_hbm.at[pl.ds(qp, NIDX)], idxd, semg1)
            pltpu.make_async_copy(src_hbm.at[pl.ds(qp, NIDX)], idxs, semg0).wait()
            pltpu.make_async_copy(dst_hbm.at[pl.ds(qp, NIDX)], idxd, semg1).wait()

            pltpu.async_copy(g_hbm.at[idxs.at[0]], rows0, semg0)
            pltpu.async_copy(g_hbm.at[idxs.at[1]], rows1, semg1)

            @pl.loop(0, npairs)
            def _(kk):
                c0 = 2 * kk
                c1 = c0 + 1
                pltpu.make_async_copy(g_hbm.at[idxs.at[c0]], rows0, semg0).wait()
                pltpu.async_copy(rows0, acc.at[idxd.at[c0]], sems0, add=True)
                pltpu.make_async_copy(g_hbm.at[idxs.at[c1]], rows1, semg1).wait()
                pltpu.async_copy(rows1, acc.at[idxd.at[c1]], sems1, add=True)
                pltpu.make_async_copy(rows0, acc.at[idxd.at[c0]], sems0).wait()
                pltpu.async_copy(g_hbm.at[idxs.at[c0 + 2]], rows0, semg0)
                pltpu.make_async_copy(rows1, acc.at[idxd.at[c1]], sems1).wait()
                pltpu.async_copy(g_hbm.at[idxs.at[c1 + 2]], rows1, semg1)

            pltpu.make_async_copy(
                g_hbm.at[idxs.at[2 * npairs]], rows0, semg0).wait()
            pltpu.make_async_copy(
                g_hbm.at[idxs.at[2 * npairs + 1]], rows1, semg1).wait()

        plsc.subcore_barrier()

        nrounds = ROWS_PER_TILE // CHUNK
        bufs = (rows0, rows1)
        gsems = (semg0, semg1)
        ssems = (sems0, sems1)

        def _rd(j):
            r0 = sid * ROWS_PER_TILE + j * CHUNK
            return acc.at[pl.ds(r0, CHUNK)], bufs[j % 2], gsems[j % 2]

        def _wr(j):
            r0 = sid * ROWS_PER_TILE + j * CHUNK
            return bufs[j % 2], out_hbm.at[cid, pl.ds(r0, CHUNK)], ssems[j % 2]

        pltpu.async_copy(*_rd(0))
        for j in range(nrounds):
            pltpu.make_async_copy(*_rd(j)).wait()
            pltpu.async_copy(*_wr(j))
            if j + 1 < nrounds:
                if j >= 1:
                    pltpu.make_async_copy(*_wr(j - 1)).wait()
                pltpu.async_copy(*_rd(j + 1))
        pltpu.make_async_copy(*_wr(nrounds - 2)).wait()
        pltpu.make_async_copy(*_wr(nrounds - 1)).wait()

    return k(g, src2d, dst2d)


def _tc_scale_input(d0, d1, x):
    def body(d0_ref, d1_ref, x_ref, o_ref):
        dinv = lax.rsqrt(d0_ref[...] + d1_ref[...] + 1.0)
        o_ref[...] = x_ref[...] * dinv

    return pl.pallas_call(
        body,
        grid=(N // BLK,),
        in_specs=[
            pl.BlockSpec((BLK, 1), lambda i: (i, 0)),
            pl.BlockSpec((BLK, 1), lambda i: (i, 0)),
            pl.BlockSpec((BLK, D_IN), lambda i: (i, 0)),
        ],
        out_specs=pl.BlockSpec((BLK, D_IN), lambda i: (i, 0)),
        out_shape=jax.ShapeDtypeStruct((N, D_IN), jnp.float32),
    )(d0, d1, x)


def _tc_layer1(d0, d1, p, g1, W1, b1, W2):
    def body(d0_ref, d1_ref, pa_ref, pb_ref, g1_ref, w1, b1r, w2, o_ref):
        dinv = lax.rsqrt(d0_ref[...] + d1_ref[...] + 1.0)
        t = dinv * (pa_ref[0] + pb_ref[0] + g1_ref[...])
        h = jnp.dot(t, w1[...], preferred_element_type=jnp.float32) + b1r[...]
        h = jnp.maximum(h, 0.0)
        o_ref[...] = dinv * jnp.dot(h, w2[...], preferred_element_type=jnp.float32)

    return pl.pallas_call(
        body,
        grid=(N // BLK,),
        in_specs=[
            pl.BlockSpec((BLK, 1), lambda i: (i, 0)),
            pl.BlockSpec((BLK, 1), lambda i: (i, 0)),
            pl.BlockSpec((1, BLK, D_IN), lambda i: (0, i, 0)),
            pl.BlockSpec((1, BLK, D_IN), lambda i: (1, i, 0)),
            pl.BlockSpec((BLK, D_IN), lambda i: (i, 0)),
            pl.BlockSpec((D_IN, D_HID), lambda i: (0, 0)),
            pl.BlockSpec((1, D_HID), lambda i: (0, 0)),
            pl.BlockSpec((D_HID, D_OUT), lambda i: (0, 0)),
        ],
        out_specs=pl.BlockSpec((BLK, D_OUT), lambda i: (i, 0)),
        out_shape=jax.ShapeDtypeStruct((N, D_OUT), jnp.float32),
    )(d0, d1, p, p, g1, W1, b1, W2)


def _tc_layer2(d0, d1, p, g2, b2):
    def body(d0_ref, d1_ref, pa_ref, pb_ref, g2_ref, b2r, o_ref):
        dinv = lax.rsqrt(d0_ref[...] + d1_ref[...] + 1.0)
        t = dinv * (pa_ref[0] + pb_ref[0] + g2_ref[...]) + b2r[...]
        o_ref[...] = jnp.maximum(t, 0.0)

    return pl.pallas_call(
        body,
        grid=(N // BLK,),
        in_specs=[
            pl.BlockSpec((BLK, 1), lambda i: (i, 0)),
            pl.BlockSpec((BLK, 1), lambda i: (i, 0)),
            pl.BlockSpec((1, BLK, D_OUT), lambda i: (0, i, 0)),
            pl.BlockSpec((1, BLK, D_OUT), lambda i: (1, i, 0)),
            pl.BlockSpec((BLK, D_OUT), lambda i: (i, 0)),
            pl.BlockSpec((1, D_OUT), lambda i: (0, 0)),
        ],
        out_specs=pl.BlockSpec((BLK, D_OUT), lambda i: (i, 0)),
        out_shape=jax.ShapeDtypeStruct((N, D_OUT), jnp.float32),
    )(d0, d1, p, p, g2, b2)


def kernel(x, edge_index, W1, b1, W2, b2):
    src = edge_index[0].astype(jnp.int32)
    dst = edge_index[1].astype(jnp.int32)
    pad = NPADROWS * CHUNK - src.shape[0]
    src2d = jnp.pad(src, (0, pad)).reshape(NPADROWS, CHUNK)
    trash = N + jnp.arange(pad, dtype=jnp.int32) % (NPAD - N)
    dst2d = jnp.concatenate([dst, trash]).reshape(NPADROWS, CHUNK)

    degp = _sc_degree(dst2d)
    d0 = degp[0, 0, :N].reshape(N, 1)
    d1 = degp[1, 0, :N].reshape(N, 1)

    g1 = _tc_scale_input(d0, d1, x)
    p1 = _sc_aggregate(g1, src2d, dst2d)
    g2 = _tc_layer1(d0, d1, p1, g1,
                    W1, b1.reshape(1, -1), W2)
    p2 = _sc_aggregate(g2, src2d, dst2d)
    out = _tc_layer2(d0, d1, p2, g2, b2.reshape(1, -1))
    return out


# speedup vs baseline: 2.8081x; 2.8081x over previous
import functools

import jax
import jax.numpy as jnp
from jax import lax
from jax.experimental import pallas as pl
from jax.experimental.pallas import tpu as pltpu
from jax.experimental.pallas import tpu_sc as plsc

N = 10000
D_IN = 128
D_HID = 256
D_OUT = 128

NC, NS = 2, 16
NW = NC * NS
NPAD = 10240
ROWS_PER_TILE = NPAD // NS
CHUNK = 128
NCHUNKS = 2500
WIN = 80
PHASE = 40
NIDX = PHASE + 8
NPADROWS = 2568
BLK = 400


def _sc_mesh():
    return plsc.VectorSubcoreMesh(core_axis_name="c", subcore_axis_name="s")


def _zero_rows(buf, nrows):
    z = jnp.zeros((16,), jnp.float32)

    @pl.loop(0, nrows)
    def _(i):
        @pl.loop(0, 8)
        def _(j):
            buf[i, pl.ds(j * 16, 16)] = z


def _sc_degree(dst2d):
    cp = pltpu.CompilerParams(needs_layout_passes=False)

    @functools.partial(
        pl.kernel,
        out_type=jax.ShapeDtypeStruct((NC, 1, NPAD), jnp.float32),
        mesh=_sc_mesh(),
        scratch_types=[
            pltpu.VMEM((WIN, CHUNK), jnp.int32),
            pltpu.VMEM((NPAD,), jnp.float32),
            pltpu.VMEM((ROWS_PER_TILE,), jnp.float32),
            pltpu.VMEM_SHARED((NS, NPAD), jnp.float32),
        ],
        compiler_params=cp,
    )
    def k(dst_hbm, out_hbm, idxd, loc, red, shr):
        cid = lax.axis_index("c")
        sid = lax.axis_index("s")
        wid = sid * NC + cid

        pltpu.sync_copy(dst_hbm.at[pl.ds(wid * WIN, WIN)], idxd)

        z = jnp.zeros((16,), jnp.float32)

        @pl.loop(0, NPAD // 16)
        def _(i):
            loc[pl.ds(i * 16, 16)] = z

        one = jnp.ones((16,), jnp.float32)

        @pl.loop(0, WIN)
        def _(i):
            @pl.loop(0, CHUNK // 16)
            def _(j):
                plsc.addupdate_scatter(loc, [idxd[i, pl.ds(j * 16, 16)]], one)

        pltpu.sync_copy(loc, shr.at[sid])
        plsc.subcore_barrier()

        c0 = sid * ROWS_PER_TILE
        pltpu.sync_copy(shr.at[0, pl.ds(c0, ROWS_PER_TILE)], red)

        @pl.loop(1, NS)
        def _(t):
            pltpu.sync_copy(shr.at[t, pl.ds(c0, ROWS_PER_TILE)],
                            loc.at[pl.ds(0, ROWS_PER_TILE)])

            @pl.loop(0, ROWS_PER_TILE // 16)
            def _(v):
                s16 = pl.ds(v * 16, 16)
                red[s16] = red[s16] + loc[s16]

        pltpu.sync_copy(red, out_hbm.at[cid, 0, pl.ds(c0, ROWS_PER_TILE)])

    return k(dst2d)


def _sc_aggregate(g, src2d, dst2d):
    @functools.partial(
        pl.kernel,
        out_type=jax.ShapeDtypeStruct((NC, NPAD, 128), jnp.float32),
        mesh=_sc_mesh(),
        scratch_types=[
            pltpu.VMEM((NIDX, CHUNK), jnp.int32),
            pltpu.VMEM((NIDX, CHUNK), jnp.int32),
            pltpu.VMEM((CHUNK, 128), jnp.float32),
            pltpu.VMEM((CHUNK, 128), jnp.float32),
            pltpu.VMEM_SHARED((NPAD, 128), jnp.float32),
            pltpu.SemaphoreType.DMA,
            pltpu.SemaphoreType.DMA,
            pltpu.SemaphoreType.DMA,
            pltpu.SemaphoreType.DMA,
        ],
    )
    def k(g_hbm, src_hbm, dst_hbm, out_hbm, idxs, idxd, rows0, rows1,
          acc, semg0, semg1, sems0, sems1):
        cid = lax.axis_index("c")
        sid = lax.axis_index("s")
        wid = sid * NC + cid
        q0 = wid * WIN

        _zero_rows(rows0, CHUNK)

        @pl.loop(0, ROWS_PER_TILE // CHUNK)
        def _(j):
            pltpu.async_copy(
                rows0, acc.at[pl.ds(sid * ROWS_PER_TILE + j * CHUNK, CHUNK)],
                sems0,
            )

        @pl.loop(0, ROWS_PER_TILE // CHUNK)
        def _(j):
            pltpu.make_async_copy(
                rows0, acc.at[pl.ds(sid * ROWS_PER_TILE + j * CHUNK, CHUNK)],
                sems0,
            ).wait()

        plsc.subcore_barrier()

        npairs = PHASE // 2

        @pl.loop(0, WIN // PHASE)
        def _(p):
            qp = q0 + PHASE * p
            pltpu.async_copy(src_hbm.at[pl.ds(qp, NIDX)], idxs, semg0)
            pltpu.async_copy(dst_hbm.at[pl.ds(qp, NIDX)], idxd, semg1)
            pltpu.make_async_copy(src_hbm.at[pl.ds(qp, NIDX)], idxs, semg0).wait()
            pltpu.make_async_copy(dst_hbm.at[pl.ds(qp, NIDX)], idxd, semg1).wait()

            pltpu.async_copy(g_hbm.at[idxs.at[0]], rows0, semg0)
            pltpu.async_copy(g_hbm.at[idxs.at[1]], rows1, semg1)

            @pl.loop(0, npairs)
            def _(kk):
                c0 = 2 * kk
                c1 = c0 + 1
                pltpu.make_async_copy(g_hbm.at[idxs.at[c0]], rows0, semg0).wait()
                pltpu.async_copy(rows0, acc.at[idxd.at[c0]], sems0, add=True)
                pltpu.make_async_copy(g_hbm.at[idxs.at[c1]], rows1, semg1).wait()
                pltpu.async_copy(rows1, acc.at[idxd.at[c1]], sems1, add=True)
                pltpu.make_async_copy(rows0, acc.at[idxd.at[c0]], sems0).wait()
                pltpu.async_copy(g_hbm.at[idxs.at[c0 + 2]], rows0, semg0)
                pltpu.make_async_copy(rows1, acc.at[idxd.at[c1]], sems1).wait()
                pltpu.async_copy(g_hbm.at[idxs.at[c1 + 2]], rows1, semg1)

            pltpu.make_async_copy(
                g_hbm.at[idxs.at[2 * npairs]], rows0, semg0).wait()
            pltpu.make_async_copy(
                g_hbm.at[idxs.at[2 * npairs + 1]], rows1, semg1).wait()

        plsc.subcore_barrier()

        nrounds = ROWS_PER_TILE // CHUNK
        bufs = (rows0, rows1)
        gsems = (semg0, semg1)
        ssems = (sems0, sems1)

        def _rd(j):
            r0 = sid * ROWS_PER_TILE + j * CHUNK
            return acc.at[pl.ds(r0, CHUNK)], bufs[j % 2], gsems[j % 2]

        def _wr(j):
            r0 = sid * ROWS_PER_TILE + j * CHUNK
            return bufs[j % 2], out_hbm.at[cid, pl.ds(r0, CHUNK)], ssems[j % 2]

        pltpu.async_copy(*_rd(0))
        for j in range(nrounds):
            pltpu.make_async_copy(*_rd(j)).wait()
            pltpu.async_copy(*_wr(j))
            if j + 1 < nrounds:
                if j >= 1:
                    pltpu.make_async_copy(*_wr(j - 1)).wait()
                pltpu.async_copy(*_rd(j + 1))
        pltpu.make_async_copy(*_wr(nrounds - 2)).wait()
        pltpu.make_async_copy(*_wr(nrounds - 1)).wait()

    return k(g, src2d, dst2d)


def _tc_scale_input(d0, d1, x):
    def body(d0_ref, d1_ref, x_ref, o_ref):
        dinv = lax.rsqrt(d0_ref[...] + d1_ref[...] + 1.0)
        o_ref[...] = x_ref[...] * dinv

    return pl.pallas_call(
        body,
        grid=(N // BLK,),
        in_specs=[
            pl.BlockSpec((BLK, 1), lambda i: (i, 0)),
            pl.BlockSpec((BLK, 1), lambda i: (i, 0)),
            pl.BlockSpec((BLK, D_IN), lambda i: (i, 0)),
        ],
        out_specs=pl.BlockSpec((BLK, D_IN), lambda i: (i, 0)),
        out_shape=jax.ShapeDtypeStruct((N, D_IN), jnp.float32),
    )(d0, d1, x)


def _tc_layer1(d0, d1, p, g1, W1, b1, W2):
    def body(d0_ref, d1_ref, pa_ref, pb_ref, g1_ref, w1, b1r, w2, o_ref):
        dinv = lax.rsqrt(d0_ref[...] + d1_ref[...] + 1.0)
        t = dinv * (pa_ref[0] + pb_ref[0] + g1_ref[...])
        h = jnp.dot(t, w1[...], preferred_element_type=jnp.float32) + b1r[...]
        h = jnp.maximum(h, 0.0)
        o_ref[...] = dinv * jnp.dot(h, w2[...], preferred_element_type=jnp.float32)

    return pl.pallas_call(
        body,
        grid=(N // BLK,),
        in_specs=[
            pl.BlockSpec((BLK, 1), lambda i: (i, 0)),
            pl.BlockSpec((BLK, 1), lambda i: (i, 0)),
            pl.BlockSpec((1, BLK, D_IN), lambda i: (0, i, 0)),
            pl.BlockSpec((1, BLK, D_IN), lambda i: (1, i, 0)),
            pl.BlockSpec((BLK, D_IN), lambda i: (i, 0)),
            pl.BlockSpec((D_IN, D_HID), lambda i: (0, 0)),
            pl.BlockSpec((1, D_HID), lambda i: (0, 0)),
            pl.BlockSpec((D_HID, D_OUT), lambda i: (0, 0)),
        ],
        out_specs=pl.BlockSpec((BLK, D_OUT), lambda i: (i, 0)),
        out_shape=jax.ShapeDtypeStruct((N, D_OUT), jnp.float32),
    )(d0, d1, p, p, g1, W1, b1, W2)


def _tc_layer2(d0, d1, p, g2, b2):
    def body(d0_ref, d1_ref, pa_ref, pb_ref, g2_ref, b2r, o_ref):
        dinv = lax.rsqrt(d0_ref[...] + d1_ref[...] + 1.0)
        t = dinv * (pa_ref[0] + pb_ref[0] + g2_ref[...]) + b2r[...]
        o_ref[...] = jnp.maximum(t, 0.0)

    return pl.pallas_call(
        body,
        grid=(N // BLK,),
        in_specs=[
            pl.BlockSpec((BLK, 1), lambda i: (i, 0)),
            pl.BlockSpec((BLK, 1), lambda i: (i, 0)),
            pl.BlockSpec((1, BLK, D_OUT), lambda i: (0, i, 0)),
            pl.BlockSpec((1, BLK, D_OUT), lambda i: (1, i, 0)),
            pl.BlockSpec((BLK, D_OUT), lambda i: (i, 0)),
            pl.BlockSpec((1, D_OUT), lambda i: (0, 0)),
        ],
        out_specs=pl.BlockSpec((BLK, D_OUT), lambda i: (i, 0)),
        out_shape=jax.ShapeDtypeStruct((N, D_OUT), jnp.float32),
    )(d0, d1, p, p, g2, b2)


def kernel(x, edge_index, W1, b1, W2, b2):
    src = edge_index[0].astype(jnp.int32)
    dst = edge_index[1].astype(jnp.int32)
    pad = NPADROWS * CHUNK - src.shape[0]
    srcpad = jnp.arange(pad, dtype=jnp.int32) % N
    src2d = jnp.concatenate([src, srcpad]).reshape(NPADROWS, CHUNK)
    trash = N + jnp.arange(pad, dtype=jnp.int32) % (NPAD - N)
    dst2d = jnp.concatenate([dst, trash]).reshape(NPADROWS, CHUNK)

    degp = _sc_degree(dst2d)
    d0 = degp[0, 0, :N].reshape(N, 1)
    d1 = degp[1, 0, :N].reshape(N, 1)

    g1 = _tc_scale_input(d0, d1, x)
    p1 = _sc_aggregate(g1, src2d, dst2d)
    g2 = _tc_layer1(d0, d1, p1, g1,
                    W1, b1.reshape(1, -1), W2)
    p2 = _sc_aggregate(g2, src2d, dst2d)
    out = _tc_layer2(d0, d1, p2, g2, b2.reshape(1, -1))
    return out


# speedup vs baseline: 2.8163x; 1.0029x over previous
import functools

import jax
import jax.numpy as jnp
from jax import lax
from jax.experimental import pallas as pl
from jax.experimental.pallas import tpu as pltpu
from jax.experimental.pallas import tpu_sc as plsc

N = 10000
D_IN = 128
D_HID = 256
D_OUT = 128

NC, NS = 2, 16
NW = NC * NS
NPAD = 10240
ROWS_PER_TILE = NPAD // NS
CHUNK = 128
NCHUNKS = 2500
WIN = 80
PHASE = 40
NIDX = PHASE + 8
NPADROWS = 2568
BLK = 400


def _sc_mesh():
    return plsc.VectorSubcoreMesh(core_axis_name="c", subcore_axis_name="s")


def _zero_rows(buf, nrows):
    z = jnp.zeros((16,), jnp.float32)

    @pl.loop(0, nrows)
    def _(i):
        @pl.loop(0, 8)
        def _(j):
            buf[i, pl.ds(j * 16, 16)] = z


def _sc_degree(dst2d):
    cp = pltpu.CompilerParams(needs_layout_passes=False)

    @functools.partial(
        pl.kernel,
        out_type=jax.ShapeDtypeStruct((NC, 1, NPAD), jnp.float32),
        mesh=_sc_mesh(),
        scratch_types=[
            pltpu.VMEM((WIN, CHUNK), jnp.int32),
            pltpu.VMEM((NPAD,), jnp.float32),
            pltpu.VMEM((ROWS_PER_TILE,), jnp.float32),
            pltpu.VMEM_SHARED((NS, NPAD), jnp.float32),
        ],
        compiler_params=cp,
    )
    def k(dst_hbm, out_hbm, idxd, loc, red, shr):
        cid = lax.axis_index("c")
        sid = lax.axis_index("s")
        wid = sid * NC + cid

        pltpu.sync_copy(dst_hbm.at[pl.ds(wid * WIN, WIN)], idxd)

        z = jnp.zeros((16,), jnp.float32)

        @pl.loop(0, NPAD // 16)
        def _(i):
            loc[pl.ds(i * 16, 16)] = z

        one = jnp.ones((16,), jnp.float32)

        @pl.loop(0, WIN)
        def _(i):
            @pl.loop(0, CHUNK // 16)
            def _(j):
                plsc.addupdate_scatter(loc, [idxd[i, pl.ds(j * 16, 16)]], one)

        pltpu.sync_copy(loc, shr.at[sid])
        plsc.subcore_barrier()

        c0 = sid * ROWS_PER_TILE
        pltpu.sync_copy(shr.at[0, pl.ds(c0, ROWS_PER_TILE)], red)

        @pl.loop(1, NS)
        def _(t):
            pltpu.sync_copy(shr.at[t, pl.ds(c0, ROWS_PER_TILE)],
                            loc.at[pl.ds(0, ROWS_PER_TILE)])

            @pl.loop(0, ROWS_PER_TILE // 16)
            def _(v):
                s16 = pl.ds(v * 16, 16)
                red[s16] = red[s16] + loc[s16]

        pltpu.sync_copy(red, out_hbm.at[cid, 0, pl.ds(c0, ROWS_PER_TILE)])

    return k(dst2d)


def _sc_aggregate(g, src2d, dst2d):
    @functools.partial(
        pl.kernel,
        out_type=jax.ShapeDtypeStruct((NC, NPAD, 128), jnp.float32),
        mesh=_sc_mesh(),
        scratch_types=[
            pltpu.VMEM((NIDX, CHUNK), jnp.int32),
            pltpu.VMEM((NIDX, CHUNK), jnp.int32),
            pltpu.VMEM((CHUNK, 128), jnp.float32),
            pltpu.VMEM((CHUNK, 128), jnp.float32),
            pltpu.VMEM_SHARED((NPAD, 128), jnp.float32),
            pltpu.SemaphoreType.DMA,
            pltpu.SemaphoreType.DMA,
            pltpu.SemaphoreType.DMA,
            pltpu.SemaphoreType.DMA,
        ],
    )
    def k(g_hbm, src_hbm, dst_hbm, out_hbm, idxs, idxd, rows0, rows1,
          acc, semg0, semg1, sems0, sems1):
        cid = lax.axis_index("c")
        sid = lax.axis_index("s")
        wid = sid * NC + cid
        q0 = wid * WIN

        _zero_rows(rows0, CHUNK)

        @pl.loop(0, ROWS_PER_TILE // CHUNK)
        def _(j):
            pltpu.async_copy(
                rows0, acc.at[pl.ds(sid * ROWS_PER_TILE + j * CHUNK, CHUNK)],
                sems0,
            )

        @pl.loop(0, ROWS_PER_TILE // CHUNK)
        def _(j):
            pltpu.make_async_copy(
                rows0, acc.at[pl.ds(sid * ROWS_PER_TILE + j * CHUNK, CHUNK)],
                sems0,
            ).wait()

        plsc.subcore_barrier()

        npairs = PHASE // 2

        @pl.loop(0, WIN // PHASE)
        def _(p):
            qp = q0 + PHASE * p
            pltpu.async_copy(src_hbm.at[pl.ds(qp, NIDX)], idxs, semg0)
            pltpu.async_copy(dst_hbm.at[pl.ds(qp, NIDX)], idxd, semg1)
            pltpu.make_async_copy(src_hbm.at[pl.ds(qp, NIDX)], idxs, semg0).wait()
            pltpu.make_async_copy(dst_hbm.at[pl.ds(qp, NIDX)], idxd, semg1).wait()

            pltpu.async_copy(g_hbm.at[idxs.at[0]], rows0, semg0)
            pltpu.async_copy(g_hbm.at[idxs.at[1]], rows1, semg1)

            @pl.loop(0, npairs, unroll=2)
            def _(kk):
                c0 = 2 * kk
                c1 = c0 + 1
                pltpu.make_async_copy(g_hbm.at[idxs.at[c0]], rows0, semg0).wait()
                pltpu.async_copy(rows0, acc.at[idxd.at[c0]], sems0, add=True)
                pltpu.make_async_copy(g_hbm.at[idxs.at[c1]], rows1, semg1).wait()
                pltpu.async_copy(rows1, acc.at[idxd.at[c1]], sems1, add=True)
                pltpu.make_async_copy(rows0, acc.at[idxd.at[c0]], sems0).wait()
                pltpu.async_copy(g_hbm.at[idxs.at[c0 + 2]], rows0, semg0)
                pltpu.make_async_copy(rows1, acc.at[idxd.at[c1]], sems1).wait()
                pltpu.async_copy(g_hbm.at[idxs.at[c1 + 2]], rows1, semg1)

            pltpu.make_async_copy(
                g_hbm.at[idxs.at[2 * npairs]], rows0, semg0).wait()
            pltpu.make_async_copy(
                g_hbm.at[idxs.at[2 * npairs + 1]], rows1, semg1).wait()

        plsc.subcore_barrier()

        nrounds = ROWS_PER_TILE // CHUNK
        bufs = (rows0, rows1)
        gsems = (semg0, semg1)
        ssems = (sems0, sems1)

        def _rd(j):
            r0 = sid * ROWS_PER_TILE + j * CHUNK
            return acc.at[pl.ds(r0, CHUNK)], bufs[j % 2], gsems[j % 2]

        def _wr(j):
            r0 = sid * ROWS_PER_TILE + j * CHUNK
            return bufs[j % 2], out_hbm.at[cid, pl.ds(r0, CHUNK)], ssems[j % 2]

        pltpu.async_copy(*_rd(0))
        for j in range(nrounds):
            pltpu.make_async_copy(*_rd(j)).wait()
            pltpu.async_copy(*_wr(j))
            if j + 1 < nrounds:
                if j >= 1:
                    pltpu.make_async_copy(*_wr(j - 1)).wait()
                pltpu.async_copy(*_rd(j + 1))
        pltpu.make_async_copy(*_wr(nrounds - 2)).wait()
        pltpu.make_async_copy(*_wr(nrounds - 1)).wait()

    return k(g, src2d, dst2d)


def _tc_scale_input(d0, d1, x):
    def body(d0_ref, d1_ref, x_ref, o_ref):
        dinv = lax.rsqrt(d0_ref[...] + d1_ref[...] + 1.0)
        o_ref[...] = x_ref[...] * dinv

    return pl.pallas_call(
        body,
        grid=(N // BLK,),
        in_specs=[
            pl.BlockSpec((BLK, 1), lambda i: (i, 0)),
            pl.BlockSpec((BLK, 1), lambda i: (i, 0)),
            pl.BlockSpec((BLK, D_IN), lambda i: (i, 0)),
        ],
        out_specs=pl.BlockSpec((BLK, D_IN), lambda i: (i, 0)),
        out_shape=jax.ShapeDtypeStruct((N, D_IN), jnp.float32),
    )(d0, d1, x)


def _tc_layer1(d0, d1, p, g1, W1, b1, W2):
    def body(d0_ref, d1_ref, pa_ref, pb_ref, g1_ref, w1, b1r, w2, o_ref):
        dinv = lax.rsqrt(d0_ref[...] + d1_ref[...] + 1.0)
        t = dinv * (pa_ref[0] + pb_ref[0] + g1_ref[...])
        h = jnp.dot(t, w1[...], preferred_element_type=jnp.float32) + b1r[...]
        h = jnp.maximum(h, 0.0)
        o_ref[...] = dinv * jnp.dot(h, w2[...], preferred_element_type=jnp.float32)

    return pl.pallas_call(
        body,
        grid=(N // BLK,),
        in_specs=[
            pl.BlockSpec((BLK, 1), lambda i: (i, 0)),
            pl.BlockSpec((BLK, 1), lambda i: (i, 0)),
            pl.BlockSpec((1, BLK, D_IN), lambda i: (0, i, 0)),
            pl.BlockSpec((1, BLK, D_IN), lambda i: (1, i, 0)),
            pl.BlockSpec((BLK, D_IN), lambda i: (i, 0)),
            pl.BlockSpec((D_IN, D_HID), lambda i: (0, 0)),
            pl.BlockSpec((1, D_HID), lambda i: (0, 0)),
            pl.BlockSpec((D_HID, D_OUT), lambda i: (0, 0)),
        ],
        out_specs=pl.BlockSpec((BLK, D_OUT), lambda i: (i, 0)),
        out_shape=jax.ShapeDtypeStruct((N, D_OUT), jnp.float32),
    )(d0, d1, p, p, g1, W1, b1, W2)


def _tc_layer2(d0, d1, p, g2, b2):
    def body(d0_ref, d1_ref, pa_ref, pb_ref, g2_ref, b2r, o_ref):
        dinv = lax.rsqrt(d0_ref[...] + d1_ref[...] + 1.0)
        t = dinv * (pa_ref[0] + pb_ref[0] + g2_ref[...]) + b2r[...]
        o_ref[...] = jnp.maximum(t, 0.0)

    return pl.pallas_call(
        body,
        grid=(N // BLK,),
        in_specs=[
            pl.BlockSpec((BLK, 1), lambda i: (i, 0)),
            pl.BlockSpec((BLK, 1), lambda i: (i, 0)),
            pl.BlockSpec((1, BLK, D_OUT), lambda i: (0, i, 0)),
            pl.BlockSpec((1, BLK, D_OUT), lambda i: (1, i, 0)),
            pl.BlockSpec((BLK, D_OUT), lambda i: (i, 0)),
            pl.BlockSpec((1, D_OUT), lambda i: (0, 0)),
        ],
        out_specs=pl.BlockSpec((BLK, D_OUT), lambda i: (i, 0)),
        out_shape=jax.ShapeDtypeStruct((N, D_OUT), jnp.float32),
    )(d0, d1, p, p, g2, b2)


def kernel(x, edge_index, W1, b1, W2, b2):
    src = edge_index[0].astype(jnp.int32)
    dst = edge_index[1].astype(jnp.int32)
    pad = NPADROWS * CHUNK - src.shape[0]
    srcpad = jnp.arange(pad, dtype=jnp.int32) % N
    src2d = jnp.concatenate([src, srcpad]).reshape(NPADROWS, CHUNK)
    trash = N + jnp.arange(pad, dtype=jnp.int32) % (NPAD - N)
    dst2d = jnp.concatenate([dst, trash]).reshape(NPADROWS, CHUNK)

    degp = _sc_degree(dst2d)
    d0 = degp[0, 0, :N].reshape(N, 1)
    d1 = degp[1, 0, :N].reshape(N, 1)

    g1 = _tc_scale_input(d0, d1, x)
    p1 = _sc_aggregate(g1, src2d, dst2d)
    g2 = _tc_layer1(d0, d1, p1, g1,
                    W1, b1.reshape(1, -1), W2)
    p2 = _sc_aggregate(g2, src2d, dst2d)
    out = _tc_layer2(d0, d1, p2, g2, b2.reshape(1, -1))
    return out
